# baseline (speedup 1.0000x reference)
import functools

import numpy as np
import jax
import jax.numpy as jnp
from jax import lax
from jax.experimental import pallas as pl
from jax.experimental.pallas import tpu as pltpu
from jax.experimental.pallas import tpu_sc as plsc

D_MODEL = 768
MAX_SEQ_LEN = 8192


def _pos_encoding(max_len, d_model):
    pos = np.arange(max_len, dtype=np.float32)[:, None]
    _2i = np.arange(0, d_model, 2, dtype=np.float32)
    div = np.power(10000.0, _2i / d_model)
    pe = np.zeros((max_len, d_model), dtype=np.float32)
    pe[:, 0::2] = np.sin(pos / div)
    pe[:, 1::2] = np.cos(pos / div)
    return pe


_PE = _pos_encoding(MAX_SEQ_LEN, D_MODEL)

_NC = 2
_NS = 16
_NW = _NC * _NS
_CHUNK = 32


@functools.lru_cache(maxsize=None)
def _make_kernel(B, S, D):
    total = B * S
    per_pos = S // _NW
    assert per_pos * _NW == S and per_pos % (2 * _CHUNK) == 0
    assert B % 2 == 0
    n_chunks = per_pos // _CHUNK
    n_col = D // 16
    mesh = plsc.VectorSubcoreMesh(core_axis_name="c", subcore_axis_name="s")

    @functools.partial(
        pl.kernel,
        mesh=mesh,
        out_type=jax.ShapeDtypeStruct((total, D), jnp.float32),
        scratch_types=[
            pltpu.VMEM((B, per_pos), jnp.int32),
            pltpu.VMEM((_CHUNK, D), jnp.float32),
            pltpu.VMEM((_CHUNK, D), jnp.float32),
            pltpu.VMEM((_CHUNK, D), jnp.float32),
            pltpu.VMEM((_CHUNK, D), jnp.float32),
            pltpu.SemaphoreType.DMA,
            pltpu.SemaphoreType.DMA,
            pltpu.SemaphoreType.DMA,
            pltpu.SemaphoreType.DMA,
            pltpu.SemaphoreType.DMA,
            pltpu.SemaphoreType.DMA,
        ],
    )
    def k(idx_hbm, table_hbm, pe_hbm, out_hbm, idx_v,
          row0, row1, pe0, pe1, g0, g1, w0, w1, p0, p1):
        rows = (row0, row1)
        pes = (pe0, pe1)
        gsem = (g0, g1)
        wsem = (w0, w1)
        psem = (p0, p1)

        wid = lax.axis_index("s") * _NC + lax.axis_index("c")
        pos0 = wid * per_pos
        for b in range(B):
            pltpu.sync_copy(idx_hbm.at[pl.ds(b * S + pos0, per_pos)],
                            idx_v.at[b])

        def pe_start(pc, slot):
            pltpu.async_copy(
                pe_hbm.at[pl.ds(pos0 + pc * _CHUNK, _CHUNK)], pes[slot],
                psem[slot])

        def pe_wait(slot):
            pltpu.make_async_copy(pe_hbm.at[pl.ds(0, _CHUNK)], pes[slot],
                                  psem[slot]).wait()

        def g_start(pc, b, slot):
            pltpu.async_copy(
                table_hbm.at[idx_v.at[b, pl.ds(pc * _CHUNK, _CHUNK)]],
                rows[slot], gsem[slot])

        def g_wait(pc, b, slot):
            pltpu.make_async_copy(
                table_hbm.at[idx_v.at[b, pl.ds(pc * _CHUNK, _CHUNK)]],
                rows[slot], gsem[slot]).wait()

        def w_start(pc, b, slot):
            pltpu.async_copy(
                rows[slot],
                out_hbm.at[pl.ds(b * S + pos0 + pc * _CHUNK, _CHUNK)],
                wsem[slot])

        def w_wait(slot):
            pltpu.make_async_copy(rows[slot], out_hbm.at[pl.ds(0, _CHUNK)],
                                  wsem[slot]).wait()

        pe_start(0, 0)
        g_start(0, 0, 0)

        @pl.loop(0, n_chunks, step=2)
        def _(pcp):
            for dpc in range(2):
                pc = pcp + dpc
                pe_slot = dpc
                if dpc == 0:

                    @pl.when(pcp + 1 < n_chunks)
                    def _():
                        pe_start(pcp + 1, 1)
                else:

                    @pl.when(pcp + 2 < n_chunks)
                    def _():
                        pe_start(pcp + 2, 0)

                pe_wait(pe_slot)
                for b in range(B):
                    slot = b % 2
                    g_wait(pc, b, slot)
                    if b == 0 and dpc == 0:

                        @pl.when(pcp > 0)
                        def _():
                            w_wait(1 - slot)
                    else:
                        w_wait(1 - slot)
                    if b == B - 1:
                        if dpc == 0:
                            g_start(pc + 1, 0, 1 - slot)
                        else:

                            @pl.when(pcp + 2 < n_chunks)
                            def _():
                                g_start(pcp + 2, 0, 1 - slot)
                    else:
                        g_start(pc, b + 1, 1 - slot)

                    row = rows[slot]
                    pe_buf = pes[pe_slot]

                    @pl.loop(0, _CHUNK, unroll=4)
                    def _(r):
                        for c in range(n_col):
                            sl = (r, pl.ds(c * 16, 16))
                            plsc.addupdate(row.at[sl], pe_buf[sl])

                    w_start(pc, b, slot)

        w_wait((B - 1) % 2)

    return k


@jax.jit
def kernel(x, table):
    B, S = x.shape
    D = table.shape[1]
    flat = x.reshape(B * S)
    out = _make_kernel(B, S, D)(flat, table, _PE[:S])
    return out.reshape(B, S, D)


# speedup vs baseline: 1.1795x; 1.1795x over previous
import functools

import numpy as np
import jax
import jax.numpy as jnp
from jax import lax
from jax.experimental import pallas as pl
from jax.experimental.pallas import tpu as pltpu
from jax.experimental.pallas import tpu_sc as plsc

D_MODEL = 768
MAX_SEQ_LEN = 8192


def _pos_encoding(max_len, d_model):
    pos = np.arange(max_len, dtype=np.float32)[:, None]
    _2i = np.arange(0, d_model, 2, dtype=np.float32)
    div = np.power(10000.0, _2i / d_model)
    pe = np.zeros((max_len, d_model), dtype=np.float32)
    pe[:, 0::2] = np.sin(pos / div)
    pe[:, 1::2] = np.cos(pos / div)
    return pe


_PE = _pos_encoding(MAX_SEQ_LEN, D_MODEL)

_NC = 2
_NS = 16
_NW = _NC * _NS
_CHUNK = 32


@functools.lru_cache(maxsize=None)
def _make_kernel(B, S, D):
    total = B * S
    per_pos = S // _NW
    assert per_pos * _NW == S and per_pos % _CHUNK == 0
    assert B == 4
    n_chunks = per_pos // _CHUNK
    n_col = D // 16
    mesh = plsc.VectorSubcoreMesh(core_axis_name="c", subcore_axis_name="s")

    @functools.partial(
        pl.kernel,
        mesh=mesh,
        out_type=jax.ShapeDtypeStruct((total, D), jnp.float32),
        scratch_types=[
            pltpu.VMEM((B, per_pos), jnp.int32),
            pltpu.VMEM((_CHUNK, D), jnp.float32),
            pltpu.VMEM((_CHUNK, D), jnp.float32),
            pltpu.VMEM((_CHUNK, D), jnp.float32),
            pltpu.VMEM((_CHUNK, D), jnp.float32),
            pltpu.VMEM((_CHUNK, D), jnp.float32),
            pltpu.SemaphoreType.DMA,
            pltpu.SemaphoreType.DMA,
            pltpu.SemaphoreType.DMA,
            pltpu.SemaphoreType.DMA,
            pltpu.SemaphoreType.DMA,
            pltpu.SemaphoreType.DMA,
            pltpu.SemaphoreType.DMA,
            pltpu.SemaphoreType.DMA,
            pltpu.SemaphoreType.DMA,
        ],
    )
    def k(idx_hbm, table_hbm, pe_hbm, out_hbm, idx_v,
          row0, row1, row2, row3, pe_buf,
          g0, g1, g2, g3, w0, w1, w2, w3, psem):
        rows = (row0, row1, row2, row3)
        gsem = (g0, g1, g2, g3)
        wsem = (w0, w1, w2, w3)

        wid = lax.axis_index("s") * _NC + lax.axis_index("c")
        pos0 = wid * per_pos
        for b in range(B):
            pltpu.sync_copy(idx_hbm.at[pl.ds(b * S + pos0, per_pos)],
                            idx_v.at[b])

        def pe_start(pc):
            pltpu.async_copy(
                pe_hbm.at[pl.ds(pos0 + pc * _CHUNK, _CHUNK)], pe_buf, psem)

        def pe_wait():
            pltpu.make_async_copy(pe_hbm.at[pl.ds(0, _CHUNK)], pe_buf,
                                  psem).wait()

        def g_start(pc, b):
            pltpu.async_copy(
                table_hbm.at[idx_v.at[b, pl.ds(pc * _CHUNK, _CHUNK)]],
                rows[b], gsem[b])

        def g_wait(pc, b):
            pltpu.make_async_copy(
                table_hbm.at[idx_v.at[b, pl.ds(pc * _CHUNK, _CHUNK)]],
                rows[b], gsem[b]).wait()

        def w_start(pc, b):
            pltpu.async_copy(
                rows[b],
                out_hbm.at[pl.ds(b * S + pos0 + pc * _CHUNK, _CHUNK)],
                wsem[b])

        def w_wait(b):
            pltpu.make_async_copy(rows[b], out_hbm.at[pl.ds(0, _CHUNK)],
                                  wsem[b]).wait()

        pe_start(0)
        g_start(0, 0)
        g_start(0, 1)
        g_start(0, 2)

        @pl.loop(0, n_chunks)
        def _(pc):
            for b in range(B):
                g_wait(pc, b)
                if b == 0:
                    pe_wait()

                @pl.loop(0, _CHUNK)
                def _(r):
                    for c in range(n_col):
                        sl = (r, pl.ds(c * 16, 16))
                        plsc.addupdate(rows[b].at[sl], pe_buf[sl])

                if b == B - 1:
                    @pl.when(pc + 1 < n_chunks)
                    def _():
                        pe_start(pc + 1)

                nb = (b + 3) % B
                npc = pc + (1 if b >= 1 else 0)
                if b == 0:

                    @pl.when(pc > 0)
                    def _():
                        w_wait(nb)
                else:
                    w_wait(nb)

                @pl.when(npc < n_chunks)
                def _():
                    g_start(npc, nb)

                w_start(pc, b)

        w_wait(B - 1)

    return k


@jax.jit
def kernel(x, table):
    B, S = x.shape
    D = table.shape[1]
    flat = x.reshape(B * S)
    out = _make_kernel(B, S, D)(flat, table, _PE[:S])
    return out.reshape(B, S, D)


# speedup vs baseline: 1.3584x; 1.1517x over previous
import functools

import numpy as np
import jax
import jax.numpy as jnp
from jax import lax
from jax.experimental import pallas as pl
from jax.experimental.pallas import tpu as pltpu
from jax.experimental.pallas import tpu_sc as plsc

D_MODEL = 768
MAX_SEQ_LEN = 8192


def _pos_encoding(max_len, d_model):
    pos = np.arange(max_len, dtype=np.float32)[:, None]
    _2i = np.arange(0, d_model, 2, dtype=np.float32)
    div = np.power(10000.0, _2i / d_model)
    pe = np.zeros((max_len, d_model), dtype=np.float32)
    pe[:, 0::2] = np.sin(pos / div)
    pe[:, 1::2] = np.cos(pos / div)
    return pe


_PE = _pos_encoding(MAX_SEQ_LEN, D_MODEL)

_NC = 2
_NS = 16
_NW = _NC * _NS
_CHUNK = 16


@functools.lru_cache(maxsize=None)
def _make_kernel(B, S, D):
    total = B * S
    per_pos = S // _NW
    assert per_pos * _NW == S and per_pos % (2 * _CHUNK) == 0
    n_chunks = per_pos // _CHUNK
    n_col = D // 16
    mesh = plsc.VectorSubcoreMesh(core_axis_name="c", subcore_axis_name="s")

    row_t = pltpu.VMEM((_CHUNK, D), jnp.float32)
    sem_t = pltpu.SemaphoreType.DMA

    @functools.partial(
        pl.kernel,
        mesh=mesh,
        out_type=jax.ShapeDtypeStruct((total, D), jnp.float32),
        scratch_types=(
            [pltpu.VMEM((B, per_pos), jnp.int32)]
            + [row_t] * (2 * (B + 1))
            + [sem_t] * (2 * (2 * B + 1))
        ),
    )
    def k(idx_hbm, table_hbm, pe_hbm, out_hbm, idx_v, *bufs_and_sems):
        n_buf = 2 * (B + 1)
        bufs, sems = bufs_and_sems[:n_buf], bufs_and_sems[n_buf:]
        rows = (bufs[:B], bufs[B + 1:2 * B + 1])
        pe_bufs = (bufs[B], bufs[2 * B + 1])
        gsem = (sems[:B], sems[B + 1:2 * B + 1])
        psem = (sems[B], sems[2 * B + 1])
        wsem = (sems[2 * B + 2:3 * B + 2], sems[3 * B + 2:])

        wid = lax.axis_index("s") * _NC + lax.axis_index("c")
        pos0 = wid * per_pos
        for b in range(B):
            pltpu.sync_copy(idx_hbm.at[pl.ds(b * S + pos0, per_pos)],
                            idx_v.at[b])

        def pe_start(pc, st):
            pltpu.async_copy(
                pe_hbm.at[pl.ds(pos0 + pc * _CHUNK, _CHUNK)], pe_bufs[st],
                psem[st])

        def pe_wait(st):
            pltpu.make_async_copy(pe_hbm.at[pl.ds(0, _CHUNK)], pe_bufs[st],
                                  psem[st]).wait()

        def g_start(pc, b, st):
            pltpu.async_copy(
                table_hbm.at[idx_v.at[b, pl.ds(pc * _CHUNK, _CHUNK)]],
                rows[st][b], gsem[st][b])

        def g_wait(pc, b, st):
            pltpu.make_async_copy(
                table_hbm.at[idx_v.at[b, pl.ds(pc * _CHUNK, _CHUNK)]],
                rows[st][b], gsem[st][b]).wait()

        def w_start(pc, b, st):
            pltpu.async_copy(
                rows[st][b],
                out_hbm.at[pl.ds(b * S + pos0 + pc * _CHUNK, _CHUNK)],
                wsem[st][b])

        def w_wait(b, st):
            pltpu.make_async_copy(rows[st][b], out_hbm.at[pl.ds(0, _CHUNK)],
                                  wsem[st][b]).wait()

        pe_start(0, 0)
        for b in range(B):
            g_start(0, b, 0)

        @pl.loop(0, n_chunks, step=2)
        def _(pcp):
            for st in range(2):
                pc = pcp + st
                nst = 1 - st
                @pl.when(pc + 1 < n_chunks)
                def _():
                    pe_start(pc + 1, nst)
                    for b in range(B):
                        if st == 0:

                            @pl.when(pcp > 0)
                            def _():
                                w_wait(b, nst)
                        else:
                            w_wait(b, nst)
                        g_start(pc + 1, b, nst)

                pe_wait(st)
                for b in range(B):
                    g_wait(pc, b, st)

                cur = rows[st]
                pe_buf = pe_bufs[st]

                @pl.loop(0, _CHUNK)
                def _(r):
                    pe_vecs = [pe_buf[r, pl.ds(c * 16, 16)]
                               for c in range(n_col)]
                    for b in range(B):
                        for c in range(n_col):
                            plsc.addupdate(
                                cur[b].at[r, pl.ds(c * 16, 16)], pe_vecs[c])

                for b in range(B):
                    w_start(pc, b, st)

        for st in range(2):
            for b in range(B):
                w_wait(b, st)

    return k


@jax.jit
def kernel(x, table):
    B, S = x.shape
    D = table.shape[1]
    flat = x.reshape(B * S)
    out = _make_kernel(B, S, D)(flat, table, _PE[:S])
    return out.reshape(B, S, D)
